# baseline (speedup 1.0000x reference)
import functools

import jax
import jax.numpy as jnp
from jax import lax
from jax.experimental import pallas as pl
from jax.experimental.pallas import tpu as pltpu
from jax.experimental.pallas import tpu_sc as plsc

B = 512
C = 17
T = 406
D = 10

NUM_CORES = 2
NUM_SUBCORES = 16
B_HALF = B // 2


def _loss_body(lab_unused, w_ref, xt_ref, pt_ref, lossT_ref, wn_ref):
    d = pl.program_id(0)

    @pl.when(d == 0)
    def _prep():
        sw = jax.nn.softplus(w_ref[...])
        wn_ref[...] = sw / jnp.sum(sw)

    sel = (lax.broadcasted_iota(jnp.int32, (1, D), 1) == d).astype(jnp.float32)
    wd_col = jnp.sum(wn_ref[...] * sel, axis=1, keepdims=True)

    xd = xt_ref[0]
    ptd = pt_ref[0]
    ad = xd * wd_col
    t2 = lax.dot_general(ptd, ad, (((1,), (0,)), ((), ())),
                         preferred_element_type=jnp.float32)
    t1 = jnp.sum(ad * xd, axis=0, keepdims=True)
    t3 = lax.dot_general(ptd * ptd, wd_col, (((1,), (0,)), ((), ())),
                         preferred_element_type=jnp.float32)
    contrib = t1 - 2.0 * t2 + t3

    @pl.when(d == 0)
    def _init():
        lossT_ref[...] = contrib

    @pl.when(d > 0)
    def _acc():
        lossT_ref[...] += contrib


@functools.cache
def _gather_sc():
    mesh = plsc.VectorSubcoreMesh(core_axis_name="c", subcore_axis_name="s")

    @functools.partial(
        pl.kernel, mesh=mesh,
        compiler_params=pltpu.CompilerParams(use_tc_tiling_on_sc=False,
                                             needs_layout_passes=False),
        out_type=jax.ShapeDtypeStruct((D, T, B), jnp.float32),
        scratch_types=[
            pltpu.VMEM((B_HALF,), jnp.int32),
            pltpu.VMEM((C, T), jnp.float32),
            pltpu.VMEM((T, B_HALF), jnp.float32),
        ],
    )
    def gather(pt_hbm, lab_hbm, out_hbm, lab_v, tab_v, buf_v):
        wid = lax.axis_index("s") * NUM_CORES + lax.axis_index("c")

        @pl.when(wid < 2 * D)
        def _work():
            d = wid // 2
            b0 = (wid - d * 2) * B_HALF
            pltpu.sync_copy(lab_hbm.at[pl.ds(b0, B_HALF)], lab_v)
            pltpu.sync_copy(pt_hbm.at[d], tab_v)

            idxs = [lab_v[pl.ds(j * 16, 16)] for j in range(B_HALF // 16)]
            zi = jnp.zeros((16,), jnp.int32)

            @plsc.parallel_loop(0, T)
            def _t_body(t):
                t16 = zi + t
                for j, idx in enumerate(idxs):
                    buf_v[t, pl.ds(j * 16, 16)] = plsc.load_gather(tab_v, [idx, t16])

            pltpu.sync_copy(buf_v, out_hbm.at[d, :, pl.ds(b0, B_HALF)])

    return gather


def kernel(input_seq, label, mask, prototypes, weights):
    xt3 = jnp.transpose(input_seq, (2, 1, 0))
    pt3 = jnp.transpose(prototypes, (2, 0, 1))
    lab2 = label.reshape(1, B)

    lossT = pl.pallas_call(
        _loss_body,
        grid=(D,),
        in_specs=[
            pl.BlockSpec((1, B), lambda d: (0, 0)),
            pl.BlockSpec((T, D), lambda d: (0, 0)),
            pl.BlockSpec((1, T, B), lambda d: (d, 0, 0)),
            pl.BlockSpec((1, C, T), lambda d: (d, 0, 0)),
        ],
        out_specs=pl.BlockSpec((C, B), lambda d: (0, 0)),
        out_shape=jax.ShapeDtypeStruct((C, B), jnp.float32),
        scratch_shapes=[pltpu.VMEM((T, D), jnp.float32)],
    )(lab2, weights, xt3, pt3)

    outT = _gather_sc()(pt3, label)

    loss = lossT.T
    output_seq = jnp.transpose(outT, (2, 1, 0))
    return (output_seq, input_seq, loss, label, label, mask)
